# baseline (speedup 1.0000x reference)
import jax
import jax.numpy as jnp
from jax import lax
from jax.experimental import pallas as pl
from jax.experimental.pallas import tpu as pltpu

_VOCAB = 1000000
_H = 64
_CTX = 200
_DEPTH = 8
_MV_BLOCK = 8192
_MV_GRID = pl.cdiv(_VOCAB, _MV_BLOCK)
_MV_LAST = _MV_GRID - 1
_MV_TAIL = _VOCAB - _MV_LAST * _MV_BLOCK
_MV_NBUF = 6


def _head_body(ids_ref, nidx_ref, pidx_ref, lidx_ref, swb_ref, tok_hbm,
               node_hbm, par_hbm, lang_hbm, nf_ref, numw_ref, numb_ref,
               hidw_ref, hidb_ref, sww_ref, hid_out, sw_out,
               buf, fbuf, sems, fsems):
    pltpu.make_async_copy(node_hbm.at[pl.ds(nidx_ref[0], 1), :],
                          fbuf.at[pl.ds(0, 1), :], fsems.at[0]).start()
    pltpu.make_async_copy(par_hbm.at[pl.ds(pidx_ref[0], 1), :],
                          fbuf.at[pl.ds(1, 1), :], fsems.at[1]).start()
    pltpu.make_async_copy(lang_hbm.at[pl.ds(lidx_ref[0], 1), :],
                          fbuf.at[pl.ds(2, 1), :], fsems.at[2]).start()

    def _issue(j, slot):
        pltpu.make_async_copy(tok_hbm.at[pl.ds(ids_ref[j], 1), :],
                              buf.at[pl.ds(slot, 1), :], sems.at[slot]).start()

    for k in range(_DEPTH):
        _issue(k, k)

    def _step(j, acc):
        slot = lax.rem(j, _DEPTH)
        pltpu.make_async_copy(tok_hbm.at[pl.ds(ids_ref[j], 1), :],
                              buf.at[pl.ds(slot, 1), :], sems.at[slot]).wait()
        acc = acc + buf[pl.ds(slot, 1), :]

        @pl.when(j + _DEPTH < _CTX)
        def _():
            _issue(j + _DEPTH, slot)

        return acc

    acc = lax.fori_loop(0, _CTX, _step, jnp.zeros((1, _H), jnp.float32))
    tok = acc * (1.0 / _CTX)

    pltpu.make_async_copy(node_hbm.at[pl.ds(nidx_ref[0], 1), :],
                          fbuf.at[pl.ds(0, 1), :], fsems.at[0]).wait()
    pltpu.make_async_copy(par_hbm.at[pl.ds(pidx_ref[0], 1), :],
                          fbuf.at[pl.ds(1, 1), :], fsems.at[1]).wait()
    pltpu.make_async_copy(lang_hbm.at[pl.ds(lidx_ref[0], 1), :],
                          fbuf.at[pl.ds(2, 1), :], fsems.at[2]).wait()

    nproj = lax.dot_general(nf_ref[...], numw_ref[...], (((1,), (1,)), ((), ())),
                            preferred_element_type=jnp.float32)
    feat = (fbuf[pl.ds(0, 1), :] + fbuf[pl.ds(1, 1), :] + fbuf[pl.ds(2, 1), :]
            + nproj + numb_ref[...])
    cat = jnp.concatenate([tok, feat], axis=1)
    hid = jnp.tanh(
        lax.dot_general(cat, hidw_ref[...], (((1,), (1,)), ((), ())),
                        preferred_element_type=jnp.float32)
        + hidb_ref[...])
    hid_out[...] = hid
    sw_out[...] = jnp.sum(hid * sww_ref[...], axis=1, keepdims=True) + swb_ref[0]


def _head(ids, nidx, pidx, lidx, token_table, node_table, parent_table,
          lang_table, nf, num_w, num_b, hid_w, hid_b, sw_w, sw_b):
    smem = pl.BlockSpec(memory_space=pltpu.SMEM)
    vmem = pl.BlockSpec(memory_space=pltpu.VMEM)
    hbm = pl.BlockSpec(memory_space=pltpu.MemorySpace.HBM)
    return pl.pallas_call(
        _head_body,
        in_specs=[smem, smem, smem, smem, smem, hbm, hbm, hbm, hbm,
                  vmem, vmem, vmem, vmem, vmem, vmem],
        out_shape=(jax.ShapeDtypeStruct((1, _H), jnp.float32),
                   jax.ShapeDtypeStruct((1, 1), jnp.float32)),
        scratch_shapes=[
            pltpu.VMEM((_DEPTH, _H), jnp.float32),
            pltpu.VMEM((4, _H), jnp.float32),
            pltpu.SemaphoreType.DMA((_DEPTH,)),
            pltpu.SemaphoreType.DMA((4,)),
        ],
    )(ids, nidx, pidx, lidx, sw_b, token_table, node_table, parent_table,
      lang_table, nf, num_w, num_b, hid_w, hid_b, sw_w)


def _mv_body(h_ref, b_ref, w_hbm, o_ref, bufs, sems):
    i = pl.program_id(0)

    def _issue(bi, slot):
        @pl.when(bi < _MV_LAST)
        def _():
            pltpu.make_async_copy(
                w_hbm.at[pl.ds(bi * _MV_BLOCK, _MV_BLOCK), :],
                bufs.at[slot], sems.at[slot]).start()

        @pl.when(bi == _MV_LAST)
        def _():
            pltpu.make_async_copy(
                w_hbm.at[pl.ds(bi * _MV_BLOCK, _MV_TAIL), :],
                bufs.at[slot, pl.ds(0, _MV_TAIL), :], sems.at[slot]).start()

    @pl.when(i == 0)
    def _():
        for k in range(_MV_NBUF - 1):
            _issue(k, k)

    nxt = i + _MV_NBUF - 1

    @pl.when(nxt < _MV_GRID)
    def _():
        _issue(nxt, lax.rem(nxt, _MV_NBUF))

    slot = lax.rem(i, _MV_NBUF)

    @pl.when(i < _MV_LAST)
    def _():
        pltpu.make_async_copy(w_hbm.at[pl.ds(i * _MV_BLOCK, _MV_BLOCK), :],
                              bufs.at[slot], sems.at[slot]).wait()

    @pl.when(i == _MV_LAST)
    def _():
        pltpu.make_async_copy(w_hbm.at[pl.ds(i * _MV_BLOCK, _MV_TAIL), :],
                              bufs.at[slot, pl.ds(0, _MV_TAIL), :],
                              sems.at[slot]).wait()
    w = bufs[slot]
    res = lax.dot_general(h_ref[...], w, (((1,), (1,)), ((), ())),
                          preferred_element_type=jnp.float32)
    o_ref[...] = res[0, :] + b_ref[...]


def _matvec(hidden, pref_w, pref_b):
    return pl.pallas_call(
        _mv_body,
        grid=(_MV_GRID,),
        in_specs=[
            pl.BlockSpec((1, _H), lambda i: (0, 0)),
            pl.BlockSpec((_MV_BLOCK,), lambda i: (i,)),
            pl.BlockSpec(memory_space=pltpu.MemorySpace.HBM),
        ],
        out_specs=pl.BlockSpec((_MV_BLOCK,), lambda i: (i,)),
        out_shape=jax.ShapeDtypeStruct((_VOCAB,), jnp.float32),
        scratch_shapes=[
            pltpu.VMEM((_MV_NBUF, _MV_BLOCK, _H), jnp.float32),
            pltpu.SemaphoreType.DMA((_MV_NBUF,)),
        ],
    )(hidden, pref_b, pref_w)


def kernel(prefix_ids, node_idx, parent_idx, lang_idx, numeric_features,
           token_table, node_table, parent_table, lang_table,
           num_W, num_b, hid_W, hid_b, sw_W, sw_b, pref_W, pref_b):
    ids = prefix_ids[-_CTX:].astype(jnp.int32)
    nidx = jnp.asarray(node_idx, jnp.int32).reshape(1)
    pidx = jnp.asarray(parent_idx, jnp.int32).reshape(1)
    lidx = jnp.asarray(lang_idx, jnp.int32).reshape(1)
    hidden, sw = _head(
        ids, nidx, pidx, lidx, token_table, node_table, parent_table,
        lang_table, numeric_features.reshape(1, 3), num_W,
        num_b.reshape(1, _H), hid_W, hid_b.reshape(1, _H), sw_W,
        sw_b.reshape(1))
    logits = _matvec(hidden, pref_W, pref_b)
    return sw[0, 0], logits


# speedup vs baseline: 1.0050x; 1.0050x over previous
import jax
import jax.numpy as jnp
from jax import lax
from jax.experimental import pallas as pl
from jax.experimental.pallas import tpu as pltpu

_VOCAB = 1000000
_H = 64
_CTX = 200
_DEPTH = 8
_MV_BLOCK = 8192
_MV_GRID = pl.cdiv(_VOCAB, _MV_BLOCK)
_MV_LAST = _MV_GRID - 1
_MV_TAIL = _VOCAB - _MV_LAST * _MV_BLOCK
_MV_NBUF = 6


def _body(ids_ref, nidx_ref, pidx_ref, lidx_ref, swb_ref, tok_hbm, node_hbm,
          par_hbm, lang_hbm, nf_ref, numw_ref, numb_ref, hidw_ref, hidb_ref,
          sww_ref, b_ref, w_hbm, sw_out, o_ref, hid_scr, buf, fbuf, bufs,
          gsems, fsems, wsems):
    i = pl.program_id(0)

    def _issue_w(bi, slot):
        @pl.when(bi < _MV_LAST)
        def _():
            pltpu.make_async_copy(
                w_hbm.at[pl.ds(bi * _MV_BLOCK, _MV_BLOCK), :],
                bufs.at[slot], wsems.at[slot]).start()

        @pl.when(bi == _MV_LAST)
        def _():
            pltpu.make_async_copy(
                w_hbm.at[pl.ds(bi * _MV_BLOCK, _MV_TAIL), :],
                bufs.at[slot, pl.ds(0, _MV_TAIL), :], wsems.at[slot]).start()

    @pl.when(i == 0)
    def _():
        for k in range(_MV_NBUF):
            _issue_w(k, k)

        pltpu.make_async_copy(node_hbm.at[pl.ds(nidx_ref[0], 1), :],
                              fbuf.at[pl.ds(0, 1), :], fsems.at[0]).start()
        pltpu.make_async_copy(par_hbm.at[pl.ds(pidx_ref[0], 1), :],
                              fbuf.at[pl.ds(1, 1), :], fsems.at[1]).start()
        pltpu.make_async_copy(lang_hbm.at[pl.ds(lidx_ref[0], 1), :],
                              fbuf.at[pl.ds(2, 1), :], fsems.at[2]).start()

        def _issue_row(j, slot):
            pltpu.make_async_copy(
                tok_hbm.at[pl.ds(ids_ref[j], 1), :],
                buf.at[pl.ds(slot, 1), :], gsems.at[slot]).start()

        for k in range(_DEPTH):
            _issue_row(k, k)

        def _step(j, acc):
            slot = lax.rem(j, _DEPTH)
            pltpu.make_async_copy(
                tok_hbm.at[pl.ds(ids_ref[j], 1), :],
                buf.at[pl.ds(slot, 1), :], gsems.at[slot]).wait()
            acc = acc + buf[pl.ds(slot, 1), :]

            @pl.when(j + _DEPTH < _CTX)
            def _():
                _issue_row(j + _DEPTH, slot)

            return acc

        acc = lax.fori_loop(0, _CTX, _step, jnp.zeros((1, _H), jnp.float32))
        tok = acc * (1.0 / _CTX)

        pltpu.make_async_copy(node_hbm.at[pl.ds(nidx_ref[0], 1), :],
                              fbuf.at[pl.ds(0, 1), :], fsems.at[0]).wait()
        pltpu.make_async_copy(par_hbm.at[pl.ds(pidx_ref[0], 1), :],
                              fbuf.at[pl.ds(1, 1), :], fsems.at[1]).wait()
        pltpu.make_async_copy(lang_hbm.at[pl.ds(lidx_ref[0], 1), :],
                              fbuf.at[pl.ds(2, 1), :], fsems.at[2]).wait()

        nproj = lax.dot_general(nf_ref[...], numw_ref[...],
                                (((1,), (1,)), ((), ())),
                                preferred_element_type=jnp.float32)
        feat = (fbuf[pl.ds(0, 1), :] + fbuf[pl.ds(1, 1), :]
                + fbuf[pl.ds(2, 1), :] + nproj + numb_ref[...])
        cat = jnp.concatenate([tok, feat], axis=1)
        hid = jnp.tanh(
            lax.dot_general(cat, hidw_ref[...], (((1,), (1,)), ((), ())),
                            preferred_element_type=jnp.float32)
            + hidb_ref[...])
        hid_scr[...] = hid
        sw_out[...] = (jnp.sum(hid * sww_ref[...], axis=1, keepdims=True)
                       + swb_ref[0])

    @pl.when(i > 0)
    def _():
        bi = i - 1
        nxt = bi + _MV_NBUF

        @pl.when(nxt < _MV_GRID)
        def _():
            _issue_w(nxt, lax.rem(nxt, _MV_NBUF))

        slot = lax.rem(bi, _MV_NBUF)

        @pl.when(bi < _MV_LAST)
        def _():
            pltpu.make_async_copy(
                w_hbm.at[pl.ds(bi * _MV_BLOCK, _MV_BLOCK), :],
                bufs.at[slot], wsems.at[slot]).wait()

        @pl.when(bi == _MV_LAST)
        def _():
            pltpu.make_async_copy(
                w_hbm.at[pl.ds(bi * _MV_BLOCK, _MV_TAIL), :],
                bufs.at[slot, pl.ds(0, _MV_TAIL), :], wsems.at[slot]).wait()

        h = hid_scr[...]
        for k in range(_MV_NBUF):
            @pl.when(slot == k)
            def _(k=k):
                res = lax.dot_general(h, bufs[k], (((1,), (1,)), ((), ())),
                                      preferred_element_type=jnp.float32)
                o_ref[...] = res[0, :] + b_ref[...]


def kernel(prefix_ids, node_idx, parent_idx, lang_idx, numeric_features,
           token_table, node_table, parent_table, lang_table,
           num_W, num_b, hid_W, hid_b, sw_W, sw_b, pref_W, pref_b):
    ids = prefix_ids[-_CTX:].astype(jnp.int32)
    nidx = jnp.asarray(node_idx, jnp.int32).reshape(1)
    pidx = jnp.asarray(parent_idx, jnp.int32).reshape(1)
    lidx = jnp.asarray(lang_idx, jnp.int32).reshape(1)
    smem = pl.BlockSpec(memory_space=pltpu.SMEM)
    vmem = pl.BlockSpec(memory_space=pltpu.VMEM)
    hbm = pl.BlockSpec(memory_space=pltpu.MemorySpace.HBM)
    mv_blk = lambda i: (jnp.maximum(i - 1, 0),)
    sw, logits = pl.pallas_call(
        _body,
        grid=(_MV_GRID + 1,),
        in_specs=[smem, smem, smem, smem, smem, hbm, hbm, hbm, hbm,
                  vmem, vmem, vmem, vmem, vmem, vmem,
                  pl.BlockSpec((_MV_BLOCK,), mv_blk), hbm],
        out_specs=(pl.BlockSpec((1, 1), lambda i: (0, 0)),
                   pl.BlockSpec((_MV_BLOCK,), mv_blk)),
        out_shape=(jax.ShapeDtypeStruct((1, 1), jnp.float32),
                   jax.ShapeDtypeStruct((_VOCAB,), jnp.float32)),
        scratch_shapes=[
            pltpu.VMEM((1, _H), jnp.float32),
            pltpu.VMEM((_DEPTH, _H), jnp.float32),
            pltpu.VMEM((4, _H), jnp.float32),
            pltpu.VMEM((_MV_NBUF, _MV_BLOCK, _H), jnp.float32),
            pltpu.SemaphoreType.DMA((_DEPTH,)),
            pltpu.SemaphoreType.DMA((4,)),
            pltpu.SemaphoreType.DMA((_MV_NBUF,)),
        ],
    )(ids, nidx, pidx, lidx, sw_b.reshape(1), token_table, node_table,
      parent_table, lang_table, numeric_features.reshape(1, 3), num_W,
      num_b.reshape(1, _H), hid_W, hid_b.reshape(1, _H), sw_W, pref_b,
      pref_W)
    return sw[0, 0], logits


# speedup vs baseline: 1.0195x; 1.0145x over previous
import jax
import jax.numpy as jnp
from jax import lax
from jax.experimental import pallas as pl
from jax.experimental.pallas import tpu as pltpu

_VOCAB = 1000000
_H = 64
_CTX = 200
_MV_BLOCK = 16384
_MV_GRID = pl.cdiv(_VOCAB, _MV_BLOCK)


def _body(ids_ref, nidx_ref, pidx_ref, lidx_ref, swb_ref, tok_hbm, node_hbm,
          par_hbm, lang_hbm, nf_ref, numw_ref, numb_ref, hidw_ref, hidb_ref,
          sww_ref, b_ref, w_ref, sw_out, o_ref, hid_scr, buf, fbuf,
          gsem, fsems):
    i = pl.program_id(0)

    @pl.when(i == 0)
    def _():
        pltpu.make_async_copy(node_hbm.at[pl.ds(nidx_ref[0], 1), :],
                              fbuf.at[pl.ds(0, 1), :], fsems.at[0]).start()
        pltpu.make_async_copy(par_hbm.at[pl.ds(pidx_ref[0], 1), :],
                              fbuf.at[pl.ds(1, 1), :], fsems.at[1]).start()
        pltpu.make_async_copy(lang_hbm.at[pl.ds(lidx_ref[0], 1), :],
                              fbuf.at[pl.ds(2, 1), :], fsems.at[2]).start()

        def _fire(j, _):
            pltpu.make_async_copy(tok_hbm.at[pl.ds(ids_ref[j], 1), :],
                                  buf.at[pl.ds(j, 1), :], gsem).start()
            return 0

        lax.fori_loop(0, _CTX, _fire, 0)

        def _drain(j, _):
            pltpu.make_async_copy(tok_hbm.at[pl.ds(ids_ref[j], 1), :],
                                  buf.at[pl.ds(j, 1), :], gsem).wait()
            return 0

        lax.fori_loop(0, _CTX, _drain, 0)
        tok = jnp.sum(buf[...], axis=0, keepdims=True) * (1.0 / _CTX)

        pltpu.make_async_copy(node_hbm.at[pl.ds(nidx_ref[0], 1), :],
                              fbuf.at[pl.ds(0, 1), :], fsems.at[0]).wait()
        pltpu.make_async_copy(par_hbm.at[pl.ds(pidx_ref[0], 1), :],
                              fbuf.at[pl.ds(1, 1), :], fsems.at[1]).wait()
        pltpu.make_async_copy(lang_hbm.at[pl.ds(lidx_ref[0], 1), :],
                              fbuf.at[pl.ds(2, 1), :], fsems.at[2]).wait()

        nproj = lax.dot_general(nf_ref[...], numw_ref[...],
                                (((1,), (1,)), ((), ())),
                                preferred_element_type=jnp.float32)
        feat = (fbuf[pl.ds(0, 1), :] + fbuf[pl.ds(1, 1), :]
                + fbuf[pl.ds(2, 1), :] + nproj + numb_ref[...])
        cat = jnp.concatenate([tok, feat], axis=1)
        hid = jnp.tanh(
            lax.dot_general(cat, hidw_ref[...], (((1,), (1,)), ((), ())),
                            preferred_element_type=jnp.float32)
            + hidb_ref[...])
        hid_scr[...] = hid
        sw_out[...] = (jnp.sum(hid * sww_ref[...], axis=1, keepdims=True)
                       + swb_ref[0])

    @pl.when(i > 0)
    def _():
        res = lax.dot_general(hid_scr[...], w_ref[...],
                              (((1,), (1,)), ((), ())),
                              preferred_element_type=jnp.float32)
        o_ref[...] = res[0, :] + b_ref[...]


def kernel(prefix_ids, node_idx, parent_idx, lang_idx, numeric_features,
           token_table, node_table, parent_table, lang_table,
           num_W, num_b, hid_W, hid_b, sw_W, sw_b, pref_W, pref_b):
    ids = prefix_ids[-_CTX:].astype(jnp.int32)
    nidx = jnp.asarray(node_idx, jnp.int32).reshape(1)
    pidx = jnp.asarray(parent_idx, jnp.int32).reshape(1)
    lidx = jnp.asarray(lang_idx, jnp.int32).reshape(1)
    smem = pl.BlockSpec(memory_space=pltpu.SMEM)
    vmem = pl.BlockSpec(memory_space=pltpu.VMEM)
    hbm = pl.BlockSpec(memory_space=pltpu.MemorySpace.HBM)
    mv_blk = lambda i: (jnp.maximum(i - 1, 0),)
    sw, logits = pl.pallas_call(
        _body,
        grid=(_MV_GRID + 1,),
        in_specs=[smem, smem, smem, smem, smem, hbm, hbm, hbm, hbm,
                  vmem, vmem, vmem, vmem, vmem, vmem,
                  pl.BlockSpec((_MV_BLOCK,), mv_blk),
                  pl.BlockSpec((_MV_BLOCK, _H),
                               lambda i: (jnp.maximum(i - 1, 0), 0))],
        out_specs=(pl.BlockSpec((1, 1), lambda i: (0, 0)),
                   pl.BlockSpec((_MV_BLOCK,), mv_blk)),
        out_shape=(jax.ShapeDtypeStruct((1, 1), jnp.float32),
                   jax.ShapeDtypeStruct((_VOCAB,), jnp.float32)),
        scratch_shapes=[
            pltpu.VMEM((1, _H), jnp.float32),
            pltpu.VMEM((_CTX, _H), jnp.float32),
            pltpu.VMEM((4, _H), jnp.float32),
            pltpu.SemaphoreType.DMA,
            pltpu.SemaphoreType.DMA((4,)),
        ],
    )(ids, nidx, pidx, lidx, sw_b.reshape(1), token_table, node_table,
      parent_table, lang_table, numeric_features.reshape(1, 3), num_W,
      num_b.reshape(1, _H), hid_W, hid_b.reshape(1, _H), sw_W, pref_b,
      pref_W)
    return sw[0, 0], logits


# speedup vs baseline: 1.0208x; 1.0013x over previous
import jax
import jax.numpy as jnp
from jax import lax
from jax.experimental import pallas as pl
from jax.experimental.pallas import tpu as pltpu

_VOCAB = 1000000
_H = 64
_CTX = 200
_MV_BLOCK = 16384
_MV_GRID = pl.cdiv(_VOCAB, _MV_BLOCK)


def _body(meta_hbm, tok_hbm, node_hbm,
          par_hbm, lang_hbm, nf_ref, numw_ref, numb_ref, hidw_ref, hidb_ref,
          sww_ref, b_ref, w_ref, sw_out, o_ref, hid_scr, buf, fbuf, ids_ref,
          gsem, fsems, msem):
    i = pl.program_id(0)

    @pl.when(i == 0)
    def _():
        pltpu.make_async_copy(meta_hbm, ids_ref, msem).start()
        pltpu.make_async_copy(meta_hbm, ids_ref, msem).wait()
        nidx_ref = ids_ref.at[pl.ds(_CTX, 1)]
        pidx_ref = ids_ref.at[pl.ds(_CTX + 1, 1)]
        lidx_ref = ids_ref.at[pl.ds(_CTX + 2, 1)]
        pltpu.make_async_copy(node_hbm.at[pl.ds(nidx_ref[0], 1), :],
                              fbuf.at[pl.ds(0, 1), :], fsems.at[0]).start()
        pltpu.make_async_copy(par_hbm.at[pl.ds(pidx_ref[0], 1), :],
                              fbuf.at[pl.ds(1, 1), :], fsems.at[1]).start()
        pltpu.make_async_copy(lang_hbm.at[pl.ds(lidx_ref[0], 1), :],
                              fbuf.at[pl.ds(2, 1), :], fsems.at[2]).start()

        def _fire(j, _):
            pltpu.make_async_copy(tok_hbm.at[pl.ds(ids_ref[j], 1), :],
                                  buf.at[pl.ds(j, 1), :], gsem).start()
            return 0

        lax.fori_loop(0, _CTX, _fire, 0)

        def _drain(j, _):
            pltpu.make_async_copy(tok_hbm.at[pl.ds(ids_ref[j], 1), :],
                                  buf.at[pl.ds(j, 1), :], gsem).wait()
            return 0

        lax.fori_loop(0, _CTX, _drain, 0)
        tok = jnp.sum(buf[...], axis=0, keepdims=True) * (1.0 / _CTX)

        pltpu.make_async_copy(node_hbm.at[pl.ds(nidx_ref[0], 1), :],
                              fbuf.at[pl.ds(0, 1), :], fsems.at[0]).wait()
        pltpu.make_async_copy(par_hbm.at[pl.ds(pidx_ref[0], 1), :],
                              fbuf.at[pl.ds(1, 1), :], fsems.at[1]).wait()
        pltpu.make_async_copy(lang_hbm.at[pl.ds(lidx_ref[0], 1), :],
                              fbuf.at[pl.ds(2, 1), :], fsems.at[2]).wait()

        nproj = lax.dot_general(nf_ref[...], numw_ref[...],
                                (((1,), (1,)), ((), ())),
                                preferred_element_type=jnp.float32)
        feat = (fbuf[pl.ds(0, 1), :] + fbuf[pl.ds(1, 1), :]
                + fbuf[pl.ds(2, 1), :] + nproj + numb_ref[...])
        cat = jnp.concatenate([tok, feat], axis=1)
        hid = jnp.tanh(
            lax.dot_general(cat, hidw_ref[...], (((1,), (1,)), ((), ())),
                            preferred_element_type=jnp.float32)
            + hidb_ref[...])
        hid_scr[...] = hid
        sw_out[...] = jnp.sum(hid * sww_ref[...], axis=1, keepdims=True)

    @pl.when(i > 0)
    def _():
        res = lax.dot_general(hid_scr[...], w_ref[...],
                              (((1,), (1,)), ((), ())),
                              preferred_element_type=jnp.float32)
        o_ref[...] = res[0, :] + b_ref[...]


def kernel(prefix_ids, node_idx, parent_idx, lang_idx, numeric_features,
           token_table, node_table, parent_table, lang_table,
           num_W, num_b, hid_W, hid_b, sw_W, sw_b, pref_W, pref_b):
    ids = prefix_ids[-_CTX:].astype(jnp.int32)
    nidx = jnp.asarray(node_idx, jnp.int32).reshape(1)
    pidx = jnp.asarray(parent_idx, jnp.int32).reshape(1)
    lidx = jnp.asarray(lang_idx, jnp.int32).reshape(1)
    vmem = pl.BlockSpec(memory_space=pltpu.VMEM)
    hbm = pl.BlockSpec(memory_space=pltpu.MemorySpace.HBM)
    mv_blk = lambda i: (jnp.maximum(i - 1, 0),)
    sw, logits = pl.pallas_call(
        _body,
        grid=(_MV_GRID + 1,),
        in_specs=[hbm, hbm, hbm, hbm, hbm,
                  vmem, vmem, vmem, vmem, vmem, vmem,
                  pl.BlockSpec((_MV_BLOCK,), mv_blk),
                  pl.BlockSpec((_MV_BLOCK, _H),
                               lambda i: (jnp.maximum(i - 1, 0), 0))],
        out_specs=(pl.BlockSpec((1, 1), lambda i: (0, 0)),
                   pl.BlockSpec((_MV_BLOCK,), mv_blk)),
        out_shape=(jax.ShapeDtypeStruct((1, 1), jnp.float32),
                   jax.ShapeDtypeStruct((_VOCAB,), jnp.float32)),
        scratch_shapes=[
            pltpu.VMEM((1, _H), jnp.float32),
            pltpu.VMEM((_CTX, _H), jnp.float32),
            pltpu.VMEM((4, _H), jnp.float32),
            pltpu.SMEM((_CTX + 3,), jnp.int32),
            pltpu.SemaphoreType.DMA,
            pltpu.SemaphoreType.DMA((4,)),
            pltpu.SemaphoreType.DMA,
        ],
    )(jnp.concatenate([ids, nidx, pidx, lidx]), token_table, node_table,
      parent_table, lang_table, numeric_features.reshape(1, 3), num_W,
      num_b.reshape(1, _H), hid_W, hid_b.reshape(1, _H), sw_W, pref_b,
      pref_W)
    return sw[0, 0] + sw_b[0], logits
